# Initial kernel scaffold; baseline (speedup 1.0000x reference)
import jax
import jax.numpy as jnp
from jax.experimental import pallas as pl


def kernel(input_ids, my_attention_mask):
    raise NotImplementedError("write your pallas kernel here")


# speedup vs baseline: 14.3124x; 14.3124x over previous
import numpy as np
import jax
import jax.numpy as jnp
from jax.experimental import pallas as pl

_MU_P = 0.15
_MASK_ID = 103

_GUMBEL_CACHE = {}


def _gumbel_const(shape):
    out = _GUMBEL_CACHE.get(shape)
    if out is None:
        with jax.ensure_compile_time_eval():
            g = jax.random.gumbel(jax.random.key(42), shape, jnp.float32)
        out = np.asarray(g)
        _GUMBEL_CACHE[shape] = out
    return out


def _select_kernel(ids_ref, w_ref, g_ref, out_ids_ref, mask_ref, neg_ref):
    w = w_ref[...]
    g = g_ref[...]
    ids = ids_ref[...]
    nz = w > 0
    cnt = jnp.sum(nz.astype(jnp.int32), axis=-1, keepdims=True)
    k = jnp.floor(_MU_P * cnt.astype(jnp.float32)).astype(jnp.int32)
    logw = jnp.where(nz, jnp.log(jnp.maximum(w, 1e-30)), -jnp.inf)
    score = jnp.where(nz, logw + g, -jnp.inf)

    b = jax.lax.bitcast_convert_type(score, jnp.int32)
    key = jnp.where(b < 0, b ^ jnp.int32(0x7FFFFFFF), b)

    kk = jnp.maximum(k, 1)
    cpos = jnp.sum((key >= 0).astype(jnp.int32), axis=-1, keepdims=True)
    t0 = jnp.where(cpos >= kk, jnp.int32(0), jnp.int32(-2147483648))

    def body(i, t):
        bit = jnp.left_shift(jnp.int32(1), jnp.int32(30) - i)
        test = t | bit
        c = jnp.sum((key >= test).astype(jnp.int32), axis=-1, keepdims=True)
        return jnp.where(c >= kk, test, t)

    t = jax.lax.fori_loop(0, 31, body, t0)

    c_gt = jnp.sum((key > t).astype(jnp.int32), axis=-1, keepdims=True)
    need = kk - c_gt
    eq = key == t
    eq_i = eq.astype(jnp.int32)
    acc = eq_i
    d = 1
    while d < acc.shape[-1]:
        shifted = jnp.concatenate(
            [jnp.zeros(acc.shape[:-1] + (d,), jnp.int32), acc[..., :-d]],
            axis=-1)
        acc = acc + shifted
        d *= 2
    prefix = acc - eq_i
    sel = ((key > t) | (eq & (prefix < need))) & (k > 0)

    out_ids_ref[...] = jnp.where(sel, jnp.int32(_MASK_ID), ids)
    m = sel.astype(jnp.float32)
    mask_ref[...] = m
    neg_ref[...] = -m


def kernel(input_ids, my_attention_mask):
    B, J, L = input_ids.shape
    R = B * J
    ids = input_ids.reshape(R, L)
    w = my_attention_mask[..., :L].reshape(R, L)
    g = jnp.asarray(_gumbel_const((B, J, L))).reshape(R, L)
    out_ids, mask, neg = pl.pallas_call(
        _select_kernel,
        out_shape=(
            jax.ShapeDtypeStruct((R, L), ids.dtype),
            jax.ShapeDtypeStruct((R, L), jnp.float32),
            jax.ShapeDtypeStruct((R, L), jnp.float32),
        ),
    )(ids, w, g)
    return (
        out_ids.reshape(B, J, L),
        mask.reshape(B, J, L),
        neg.reshape(B, J, L),
    )
